# Initial kernel scaffold; baseline (speedup 1.0000x reference)
import functools

import jax
import jax.numpy as jnp
from jax import lax
from jax.experimental import pallas as pl
from jax.experimental.pallas import tpu as pltpu
from jax.experimental.pallas import tpu_sc as plsc

HIDDEN = 32
NUM_CORES = 2
NUM_SUBCORES = 16
NW = NUM_CORES * NUM_SUBCORES

CHUNK = 1600


@functools.partial(jax.jit, static_argnums=(2, 3))
def _gather_call(y_flat, table, B, n_chunks):
    b_per_w = B // NW
    mesh = plsc.VectorSubcoreMesh(core_axis_name="c", subcore_axis_name="s")

    def body(idx_hbm, table_hbm, out_hbm, idx_v, rows_v, sem):
        wid = lax.axis_index("s") * NUM_CORES + lax.axis_index("c")
        base = pl.multiple_of(wid * b_per_w, 8)

        def step(i, carry):
            off = pl.multiple_of(base + i * CHUNK, 8)
            pltpu.sync_copy(idx_hbm.at[pl.ds(off, CHUNK)], idx_v)
            pltpu.async_copy(table_hbm.at[idx_v], rows_v, sem).wait()
            pltpu.sync_copy(rows_v, out_hbm.at[pl.ds(off, CHUNK)])
            return carry

        lax.fori_loop(0, n_chunks, step, 0)

    run = pl.kernel(
        body,
        out_type=jax.ShapeDtypeStruct((B, HIDDEN), jnp.float32),
        mesh=mesh,
        scratch_types=[
            pltpu.VMEM((CHUNK,), jnp.int32),
            pltpu.VMEM((CHUNK, HIDDEN), jnp.float32),
            pltpu.SemaphoreType.DMA,
        ],
    )
    return run(y_flat, table)


def kernel(y, table):
    shape = y.shape
    B = y.size
    y_flat = y.reshape((B,))
    n_chunks = B // NW // CHUNK
    out = _gather_call(y_flat, table, B, n_chunks)
    return out.reshape(shape + (HIDDEN,))


# speedup vs baseline: 1.5175x; 1.5175x over previous
import functools

import jax
import jax.numpy as jnp
from jax import lax
from jax.experimental import pallas as pl
from jax.experimental.pallas import tpu as pltpu
from jax.experimental.pallas import tpu_sc as plsc

HIDDEN = 32
NUM_CORES = 2
NUM_SUBCORES = 16
NW = NUM_CORES * NUM_SUBCORES

CHUNK = 1600


@functools.partial(jax.jit, static_argnums=(2, 3))
def _gather_call(y_flat, table, B, n_chunks):
    b_per_w = B // NW
    mesh = plsc.VectorSubcoreMesh(core_axis_name="c", subcore_axis_name="s")

    def body(idx_hbm, table_hbm, out_hbm, idx_v, rows_v, sem):
        wid = lax.axis_index("s") * NUM_CORES + lax.axis_index("c")
        base = pl.multiple_of(wid * b_per_w, 8)

        def step(i, carry):
            off = pl.multiple_of(base + i * CHUNK, 8)
            pltpu.sync_copy(idx_hbm.at[pl.ds(off, CHUNK)], idx_v)
            pltpu.async_copy(table_hbm.at[idx_v], rows_v, sem).wait()
            pltpu.sync_copy(rows_v, out_hbm.at[pl.ds(off, CHUNK)])
            return carry

        lax.fori_loop(0, n_chunks, step, 0)

    run = pl.kernel(
        body,
        out_type=jax.ShapeDtypeStruct((B, HIDDEN), jnp.float32),
        mesh=mesh,
        scratch_types=[
            pltpu.VMEM((CHUNK,), jnp.int32),
            pltpu.VMEM((CHUNK, HIDDEN), jnp.float32),
            pltpu.SemaphoreType.DMA,
        ],
        compiler_params=pltpu.CompilerParams(use_tc_tiling_on_sc=False),
    )
    return run(y_flat, table)


def kernel(y, table):
    shape = y.shape
    B = y.size
    y_flat = y.reshape((B,))
    n_chunks = B // NW // CHUNK
    out = _gather_call(y_flat, table, B, n_chunks)
    return out.reshape(shape + (HIDDEN,))


# speedup vs baseline: 1.5333x; 1.0104x over previous
import functools

import jax
import jax.numpy as jnp
from jax import lax
from jax.experimental import pallas as pl
from jax.experimental.pallas import tpu as pltpu
from jax.experimental.pallas import tpu_sc as plsc

HIDDEN = 32
NUM_CORES = 2
NUM_SUBCORES = 16
NW = NUM_CORES * NUM_SUBCORES

CHUNK = 1600


@functools.partial(jax.jit, static_argnums=(2, 3))
def _gather_call(y_flat, table, B, n_chunks):
    b_per_w = B // NW
    mesh = plsc.VectorSubcoreMesh(core_axis_name="c", subcore_axis_name="s")

    def body(idx_hbm, table_hbm, out_hbm,
             idx0, idx1, rows0, rows1,
             isem0, isem1, gsem0, gsem1, ssem0, ssem1):
        idx_v = (idx0, idx1)
        rows_v = (rows0, rows1)
        isem = (isem0, isem1)
        gsem = (gsem0, gsem1)
        ssem = (ssem0, ssem1)

        wid = lax.axis_index("s") * NUM_CORES + lax.axis_index("c")
        base = pl.multiple_of(wid * b_per_w, 8)

        def chunk_off(i):
            return pl.multiple_of(base + i * CHUNK, 8)

        def start_idx(i, b):
            return pltpu.async_copy(
                idx_hbm.at[pl.ds(chunk_off(i), CHUNK)], idx_v[b], isem[b])

        def start_gather(b):
            return pltpu.async_copy(table_hbm.at[idx_v[b]], rows_v[b], gsem[b])

        def start_store(i, b):
            return pltpu.async_copy(
                rows_v[b], out_hbm.at[pl.ds(chunk_off(i), CHUNK)], ssem[b])

        h_idx = [None, None]
        h_gather = [None, None]
        h_store = [None, None]
        h_idx[0] = start_idx(0, 0)
        if n_chunks > 1:
            h_idx[1] = start_idx(1, 1)
        h_idx[0].wait()
        h_gather[0] = start_gather(0)

        for i in range(n_chunks):
            b = i & 1
            o = b ^ 1
            h_gather[b].wait()
            if i + 2 < n_chunks:
                h_idx[b] = start_idx(i + 2, b)
            h_store[b] = start_store(i, b)
            if i + 1 < n_chunks:
                h_idx[o].wait()
                if i >= 1:
                    h_store[o].wait()
                h_gather[o] = start_gather(o)

        if n_chunks > 1:
            h_store[(n_chunks - 2) & 1].wait()
        h_store[(n_chunks - 1) & 1].wait()

    run = pl.kernel(
        body,
        out_type=jax.ShapeDtypeStruct((B, HIDDEN), jnp.float32),
        mesh=mesh,
        scratch_types=[
            pltpu.VMEM((CHUNK,), jnp.int32),
            pltpu.VMEM((CHUNK,), jnp.int32),
            pltpu.VMEM((CHUNK, HIDDEN), jnp.float32),
            pltpu.VMEM((CHUNK, HIDDEN), jnp.float32),
            pltpu.SemaphoreType.DMA,
            pltpu.SemaphoreType.DMA,
            pltpu.SemaphoreType.DMA,
            pltpu.SemaphoreType.DMA,
            pltpu.SemaphoreType.DMA,
            pltpu.SemaphoreType.DMA,
        ],
        compiler_params=pltpu.CompilerParams(use_tc_tiling_on_sc=False),
    )
    return run(y_flat, table)


def kernel(y, table):
    shape = y.shape
    B = y.size
    y_flat = y.reshape((B,))
    n_chunks = B // NW // CHUNK
    out = _gather_call(y_flat, table, B, n_chunks)
    return out.reshape(shape + (HIDDEN,))


# speedup vs baseline: 1.5504x; 1.0111x over previous
import functools

import jax
import jax.numpy as jnp
from jax import lax
from jax.experimental import pallas as pl
from jax.experimental.pallas import tpu as pltpu
from jax.experimental.pallas import tpu_sc as plsc

HIDDEN = 32
NUM_CORES = 2
NUM_SUBCORES = 16
NW = NUM_CORES * NUM_SUBCORES

CHUNK = 1600


@functools.partial(jax.jit, static_argnums=(2, 3))
def _gather_call(y_flat, table, B, n_chunks):
    b_per_w = B // NW
    mesh = plsc.VectorSubcoreMesh(core_axis_name="c", subcore_axis_name="s")

    def body(idx_hbm, table_hbm, out_hbm,
             idx0, idx1, rows0, rows1,
             isem0, isem1, gsem0, gsem1, ssem0, ssem1):
        idx_v = (idx0, idx1)
        rows_v = (rows0, rows1)
        isem = (isem0, isem1)
        gsem = (gsem0, gsem1)
        ssem = (ssem0, ssem1)

        wid = lax.axis_index("s") * NUM_CORES + lax.axis_index("c")
        base = pl.multiple_of(wid * b_per_w, 8)

        def chunk_off(i):
            return pl.multiple_of(base + i * CHUNK, 8)

        def start_idx(i, b):
            return pltpu.async_copy(
                idx_hbm.at[pl.ds(chunk_off(i), CHUNK)], idx_v[b], isem[b])

        def start_gather(b):
            return pltpu.async_copy(table_hbm.at[idx_v[b]], rows_v[b], gsem[b])

        def start_store(i, b):
            return pltpu.async_copy(
                rows_v[b], out_hbm.at[pl.ds(chunk_off(i), CHUNK)], ssem[b])

        h_idx = [None, None]
        h_gather = [None, None]
        h_store = [None, None]
        h_idx[0] = start_idx(0, 0)
        if n_chunks > 1:
            h_idx[1] = start_idx(1, 1)
        h_idx[0].wait()
        h_gather[0] = start_gather(0)

        for i in range(n_chunks):
            b = i & 1
            o = b ^ 1
            h_gather[b].wait()
            if i + 2 < n_chunks:
                h_idx[b] = start_idx(i + 2, b)
            h_store[b] = start_store(i, b)
            if i + 1 < n_chunks:
                h_idx[o].wait()
                if i >= 1:
                    h_store[o].wait()
                h_gather[o] = start_gather(o)

        if n_chunks > 1:
            h_store[(n_chunks - 2) & 1].wait()
        h_store[(n_chunks - 1) & 1].wait()

    run = pl.kernel(
        body,
        out_type=jax.ShapeDtypeStruct((B, HIDDEN), jnp.float32),
        mesh=mesh,
        scratch_types=[
            pltpu.VMEM((CHUNK,), jnp.int32),
            pltpu.VMEM((CHUNK,), jnp.int32),
            pltpu.VMEM((CHUNK, HIDDEN), jnp.float32),
            pltpu.VMEM((CHUNK, HIDDEN), jnp.float32),
            pltpu.SemaphoreType.DMA,
            pltpu.SemaphoreType.DMA,
            pltpu.SemaphoreType.DMA,
            pltpu.SemaphoreType.DMA,
            pltpu.SemaphoreType.DMA,
            pltpu.SemaphoreType.DMA,
        ],
        compiler_params=pltpu.CompilerParams(use_tc_tiling_on_sc=False),
    )
    return run(y_flat, table)


def kernel(y, table):
    S, BATCH, J = y.shape
    B = y.size
    CB = BATCH // 128
    y_flat = y.reshape(S, CB, 128, J).transpose(0, 1, 3, 2).reshape(B)
    n_chunks = B // NW // CHUNK
    out = _gather_call(y_flat, table, B, n_chunks)
    return (out.reshape(S, CB, J, 128, HIDDEN)
               .transpose(0, 1, 3, 2, 4)
               .reshape(S, BATCH, J, HIDDEN))
